# baseline (speedup 1.0000x reference)
import functools

import jax
import jax.numpy as jnp
from jax import lax
from jax.experimental import pallas as pl
from jax.experimental.pallas import tpu as pltpu
from jax.experimental.pallas import tpu_sc as plsc

N, C, H, W = 4, 96, 512, 512
HW = H * W
P_TOTAL = N * HW

NC, NS, L = 2, 16, 16
NW = NC * NS
PW = P_TOTAL // NW
CH = 128
N_CHUNKS = PW // CH
WPB = NW // N


def _prep_and_fire(tab, gx, gy, gxv, gyv, idxv, wflat, rows, sem, wid, i):
    p0 = wid * PW + i * CH
    row_base = (wid // WPB) * HW
    pltpu.sync_copy(gx.at[pl.ds(p0, CH)], gxv)
    pltpu.sync_copy(gy.at[pl.ds(p0, CH)], gyv)

    for v in range(CH // L):
        sl = pl.ds(v * L, L)
        x = (gxv[sl] + 1.0) * ((W - 1) / 2.0)
        y = (gyv[sl] + 1.0) * ((H - 1) / 2.0)
        xi = jnp.minimum(jnp.maximum(x.astype(jnp.int32), 0), W - 2)
        yi = jnp.minimum(jnp.maximum(y.astype(jnp.int32), 0), H - 2)
        fx = x - xi.astype(jnp.float32)
        fy = y - yi.astype(jnp.float32)
        ra = yi * W + xi + row_base
        idxv[0, sl] = ra
        idxv[1, sl] = ra + W
        idxv[2, sl] = ra + 1
        idxv[3, sl] = ra + (W + 1)
        ex = 1.0 - fx
        ey = 1.0 - fy
        for k, wk in enumerate((ex * ey, ex * fy, fx * ey, fx * fy)):
            pk = plsc.pack(wk, wk, format=plsc.PackFormat.INTERLEAVED)
            wflat[pl.ds(k * CH + v * L, L)] = plsc.bitcast(pk, jnp.int32)

    for j in range(4):
        pltpu.async_copy(tab.at[idxv.at[j]], rows.at[j], sem)


def _combine(wflat, rows, ob):
    def pos_body(p, _):
        pv = jnp.full((L,), p, dtype=jnp.int32)
        w4 = [plsc.bitcast(plsc.load_gather(wflat, [pv + k * CH]),
                           jnp.bfloat16) for k in range(4)]
        for j in range(C // (2 * L)):
            s2 = pl.ds(j * 2 * L, 2 * L)
            ob[p, s2] = (rows[0, p, s2] * w4[0] + rows[1, p, s2] * w4[1] +
                         rows[2, p, s2] * w4[2] + rows[3, p, s2] * w4[3])
        return 0

    lax.fori_loop(0, CH, pos_body, 0, unroll=False)


def _sampler_body(tab, gx, gy, out,
                  gxv, gyv, idxv0, idxv1, w0, w1, rows0, rows1,
                  ob0, ob1, sem0, sem1, semo):
    wid = lax.axis_index("s") * NC + lax.axis_index("c")
    base = wid * PW

    fire = functools.partial(_prep_and_fire, tab, gx, gy, gxv, gyv)

    fire(idxv0, w0, rows0, sem0, wid, 0)
    fire(idxv1, w1, rows1, sem1, wid, 1)

    def body(k, _):
        c0 = 2 * k

        for j in range(4):
            pltpu.make_async_copy(tab.at[idxv0.at[j]], rows0.at[j],
                                  sem0).wait()
        _combine(w0, rows0, ob0)
        out_cp0 = pltpu.async_copy(
            ob0, out.at[pl.ds(base + c0 * CH, CH)], semo)
        pl.when(c0 + 2 < N_CHUNKS)(
            lambda: fire(idxv0, w0, rows0, sem0, wid, c0 + 2))

        for j in range(4):
            pltpu.make_async_copy(tab.at[idxv1.at[j]], rows1.at[j],
                                  sem1).wait()
        _combine(w1, rows1, ob1)
        out_cp1 = pltpu.async_copy(
            ob1, out.at[pl.ds(base + (c0 + 1) * CH, CH)], semo)
        pl.when(c0 + 3 < N_CHUNKS)(
            lambda: fire(idxv1, w1, rows1, sem1, wid, c0 + 3))

        out_cp0.wait()
        out_cp1.wait()
        return 0

    lax.fori_loop(0, N_CHUNKS // 2, body, 0, unroll=False)


_sampler = pl.kernel(
    _sampler_body,
    out_type=jax.ShapeDtypeStruct((P_TOTAL, C), jnp.bfloat16),
    mesh=plsc.VectorSubcoreMesh(core_axis_name="c", subcore_axis_name="s"),
    scratch_types=[
        pltpu.VMEM((CH,), jnp.float32),
        pltpu.VMEM((CH,), jnp.float32),
        pltpu.VMEM((4, CH), jnp.int32),
        pltpu.VMEM((4, CH), jnp.int32),
        pltpu.VMEM((4 * CH,), jnp.int32),
        pltpu.VMEM((4 * CH,), jnp.int32),
        pltpu.VMEM((4, CH, C), jnp.bfloat16),
        pltpu.VMEM((4, CH, C), jnp.bfloat16),
        pltpu.VMEM((CH, C), jnp.bfloat16),
        pltpu.VMEM((CH, C), jnp.bfloat16),
        pltpu.SemaphoreType.DMA,
        pltpu.SemaphoreType.DMA,
        pltpu.SemaphoreType.DMA,
    ],
    compiler_params=pltpu.CompilerParams(
        needs_layout_passes=False, use_tc_tiling_on_sc=False),
)


def kernel(tenInput, g):
    tab = jnp.transpose(tenInput.astype(jnp.bfloat16),
                        (0, 2, 3, 1)).reshape(P_TOTAL, C)
    gx = g[..., 0].reshape(P_TOTAL)
    gy = g[..., 1].reshape(P_TOTAL)
    out = _sampler(tab, gx, gy).astype(jnp.float32)
    return out.reshape(N, H, W, C).transpose(0, 3, 1, 2)


# speedup vs baseline: 1.3693x; 1.3693x over previous
import functools

import jax
import jax.numpy as jnp
from jax import lax
from jax.experimental import pallas as pl
from jax.experimental.pallas import tpu as pltpu
from jax.experimental.pallas import tpu_sc as plsc

N, C, H, W = 4, 96, 512, 512
HW = H * W
P_TOTAL = N * HW

NC, NS, L = 2, 16, 16
NW = NC * NS
PW = P_TOTAL // NW
CH = 128
N_CHUNKS = PW // CH
WPB = NW // N
GB = 16


def _prep_and_fire(tab, gxv, gyv, idxv, wflat, rows, sem, wid, i):
    row_base = (wid // WPB) * HW
    goff = lax.rem(i, GB) * CH

    for v in range(CH // L):
        sl = pl.ds(goff + v * L, L)
        x = (gxv[sl] + 1.0) * ((W - 1) / 2.0)
        y = (gyv[sl] + 1.0) * ((H - 1) / 2.0)
        sl = pl.ds(v * L, L)
        xi = jnp.minimum(jnp.maximum(x.astype(jnp.int32), 0), W - 2)
        yi = jnp.minimum(jnp.maximum(y.astype(jnp.int32), 0), H - 2)
        fx = x - xi.astype(jnp.float32)
        fy = y - yi.astype(jnp.float32)
        ra = yi * W + xi + row_base
        idxv[0, sl] = ra
        idxv[1, sl] = ra + W
        idxv[2, sl] = ra + 1
        idxv[3, sl] = ra + (W + 1)
        ex = 1.0 - fx
        ey = 1.0 - fy
        wflat[pl.ds(v * L, L)] = ex * ey
        wflat[pl.ds(CH + v * L, L)] = ex * fy
        wflat[pl.ds(2 * CH + v * L, L)] = fx * ey
        wflat[pl.ds(3 * CH + v * L, L)] = fx * fy

    for j in range(4):
        pltpu.async_copy(tab.at[idxv.at[j]], rows.at[j], sem)


def _combine(wflat, rows, ob):
    def pos_body(p, _):
        pv = jnp.full((L,), p, dtype=jnp.int32)
        wa = plsc.load_gather(wflat, [pv])
        wb = plsc.load_gather(wflat, [pv + CH])
        wc = plsc.load_gather(wflat, [pv + 2 * CH])
        wd = plsc.load_gather(wflat, [pv + 3 * CH])
        for j in range(C // L):
            s2 = pl.ds(j * L, L)
            ob[p, s2] = (rows[0, p, s2] * wa + rows[1, p, s2] * wb +
                         rows[2, p, s2] * wc + rows[3, p, s2] * wd)
        return 0

    lax.fori_loop(0, CH, pos_body, 0, unroll=False)


def _sampler_body(tab, gx, gy, out,
                  gxv, gyv, idxv0, idxv1, w0, w1, rows0, rows1,
                  ob0, ob1, sem0, sem1, semo0, semo1):
    wid = lax.axis_index("s") * NC + lax.axis_index("c")
    base = wid * PW

    fire = functools.partial(_prep_and_fire, tab, gxv, gyv)

    def load_grid_block(b):
        p0 = base + b * (GB * CH)
        pltpu.sync_copy(gx.at[pl.ds(p0, GB * CH)], gxv)
        pltpu.sync_copy(gy.at[pl.ds(p0, GB * CH)], gyv)

    load_grid_block(0)
    fire(idxv0, w0, rows0, sem0, wid, 0)
    fire(idxv1, w1, rows1, sem1, wid, 1)

    def body(k, _):
        c0 = 2 * k

        pl.when(jnp.logical_and(lax.rem(c0 + 2, GB) == 0,
                                c0 + 2 < N_CHUNKS))(
            lambda: load_grid_block((c0 + 2) // GB))

        for j in range(4):
            pltpu.make_async_copy(tab.at[idxv0.at[j]], rows0.at[j],
                                  sem0).wait()
        pl.when(k > 0)(
            lambda: pltpu.make_async_copy(
                ob0, out.at[pl.ds(base, CH)], semo0).wait())
        _combine(w0, rows0, ob0)
        pltpu.async_copy(ob0, out.at[pl.ds(base + c0 * CH, CH)], semo0)
        pl.when(c0 + 2 < N_CHUNKS)(
            lambda: fire(idxv0, w0, rows0, sem0, wid, c0 + 2))

        for j in range(4):
            pltpu.make_async_copy(tab.at[idxv1.at[j]], rows1.at[j],
                                  sem1).wait()
        pl.when(k > 0)(
            lambda: pltpu.make_async_copy(
                ob1, out.at[pl.ds(base, CH)], semo1).wait())
        _combine(w1, rows1, ob1)
        pltpu.async_copy(ob1, out.at[pl.ds(base + (c0 + 1) * CH, CH)], semo1)
        pl.when(c0 + 3 < N_CHUNKS)(
            lambda: fire(idxv1, w1, rows1, sem1, wid, c0 + 3))
        return 0

    lax.fori_loop(0, N_CHUNKS // 2, body, 0, unroll=False)

    pltpu.make_async_copy(ob0, out.at[pl.ds(base, CH)], semo0).wait()
    pltpu.make_async_copy(ob1, out.at[pl.ds(base, CH)], semo1).wait()


_sampler = pl.kernel(
    _sampler_body,
    out_type=jax.ShapeDtypeStruct((P_TOTAL, C), jnp.float32),
    mesh=plsc.VectorSubcoreMesh(core_axis_name="c", subcore_axis_name="s"),
    scratch_types=[
        pltpu.VMEM((GB * CH,), jnp.float32),
        pltpu.VMEM((GB * CH,), jnp.float32),
        pltpu.VMEM((4, CH), jnp.int32),
        pltpu.VMEM((4, CH), jnp.int32),
        pltpu.VMEM((4 * CH,), jnp.float32),
        pltpu.VMEM((4 * CH,), jnp.float32),
        pltpu.VMEM((4, CH, C), jnp.float32),
        pltpu.VMEM((4, CH, C), jnp.float32),
        pltpu.VMEM((CH, C), jnp.float32),
        pltpu.VMEM((CH, C), jnp.float32),
        pltpu.SemaphoreType.DMA,
        pltpu.SemaphoreType.DMA,
        pltpu.SemaphoreType.DMA,
        pltpu.SemaphoreType.DMA,
    ],
    compiler_params=pltpu.CompilerParams(
        needs_layout_passes=False, use_tc_tiling_on_sc=False),
)


def kernel(tenInput, g):
    tab = jnp.transpose(tenInput, (0, 2, 3, 1)).reshape(P_TOTAL, C)
    gx = g[..., 0].reshape(P_TOTAL)
    gy = g[..., 1].reshape(P_TOTAL)
    out = _sampler(tab, gx, gy)
    return out.reshape(N, H, W, C).transpose(0, 3, 1, 2)
